# baseline (speedup 1.0000x reference)
import functools

import numpy as np
import jax
import jax.numpy as jnp
from jax import lax
from jax.experimental import pallas as pl
from jax.experimental.pallas import tpu as pltpu
from jax.experimental.pallas import tpu_sc as plsc

H = 1024
Hh = 512
NB = 60000
NBP = 61440
RPW = 1920
G = 8
DEPTH = 12
BM = 512
HALF = NBP // 2
T = 31
BATCH = 16
TB = T * BATCH


def _elu(x):
    return jnp.where(x > 0, x, jnp.exp(jnp.minimum(x, 0.0)) - 1.0)


def _make_gather3(n_pad, rpw, g):
    nblk = rpw // g
    assert g % 8 == 0
    mesh = plsc.VectorSubcoreMesh(core_axis_name="c", subcore_axis_name="s")

    def _adds(r0, r1, r2):
        def row(j, _):
            for cb in range(H // 16):
                s = cb * 16
                r0[j, pl.ds(s, 16)] = (r0[j, pl.ds(s, 16)]
                                       + r1[j, pl.ds(s, 16)]
                                       + r2[j, pl.ds(s, 16)])
            return 0
        lax.fori_loop(0, g, row, 0)

    if nblk == 1:
        @functools.partial(
            pl.kernel, mesh=mesh,
            out_type=jax.ShapeDtypeStruct((n_pad, H), jnp.float32),
            scratch_types=[pltpu.VMEM((3, rpw), jnp.int32)]
            + [pltpu.VMEM((g, H), jnp.float32)] * 3
            + [pltpu.SemaphoreType.DMA],
        )
        def gather3_small(table_hbm, idx_hbm, out_hbm, idx_v, r0, r1, r2, sem):
            wid = lax.axis_index("s") * 2 + lax.axis_index("c")
            pltpu.sync_copy(idx_hbm.at[wid], idx_v)
            for k, r in ((0, r0), (1, r1), (2, r2)):
                pltpu.async_copy(table_hbm.at[idx_v.at[k, pl.ds(0, g)]], r, sem)
            for k, r in ((0, r0), (1, r1), (2, r2)):
                pltpu.make_async_copy(table_hbm.at[idx_v.at[k, pl.ds(0, g)]], r, sem).wait()
            _adds(r0, r1, r2)
            pltpu.sync_copy(r0, out_hbm.at[pl.ds(wid * rpw, g)])

        return gather3_small

    NS = 4
    assert nblk % NS == 0

    @functools.partial(
        pl.kernel, mesh=mesh,
        out_type=jax.ShapeDtypeStruct((n_pad, H), jnp.float32),
        scratch_types=[pltpu.VMEM((3, rpw), jnp.int32)]
        + [pltpu.VMEM((g, H), jnp.float32)] * (3 * NS)
        + [pltpu.SemaphoreType.DMA] * (2 * NS),
    )
    def gather3(table_hbm, idx_hbm, tok_hbm, out_hbm, idx_v, *bufsem):
        del tok_hbm
        wid = lax.axis_index("s") * 2 + lax.axis_index("c")
        pltpu.sync_copy(idx_hbm.at[wid], idx_v)
        rs = bufsem[:3 * NS]
        sems = bufsem[3 * NS:]
        bufs = tuple((rs[3 * s], rs[3 * s + 1], rs[3 * s + 2],
                      sems[2 * s], sems[2 * s + 1]) for s in range(NS))

        def wait_out(s):
            r0, _, _, _, osem = bufs[s]
            pltpu.make_async_copy(r0, out_hbm.at[pl.ds(wid * rpw, g)], osem).wait()

        def fire(j, s):
            @pl.when(j >= NS)
            def _():
                wait_out(s)

            r0, r1, r2, sem, _ = bufs[s]
            for k, r in ((0, r0), (1, r1), (2, r2)):
                pltpu.async_copy(table_hbm.at[idx_v.at[k, pl.ds(j * g, g)]], r, sem)

        def emit(bi, s):
            r0, r1, r2, sem, osem = bufs[s]
            for k, r in ((0, r0), (1, r1), (2, r2)):
                pltpu.make_async_copy(table_hbm.at[idx_v.at[k, pl.ds(0, g)]], r, sem).wait()
            _adds(r0, r1, r2)
            pltpu.async_copy(r0, out_hbm.at[pl.ds(wid * rpw + bi * g, g)], osem)

        for s in range(NS - 1):
            fire(s, s)

        def rnd(bq, _):
            b0 = bq * NS
            for u in range(NS):
                b = b0 + u
                j = b + NS - 1

                @pl.when(j < nblk)
                def _():
                    fire(j, (u + NS - 1) % NS)

                emit(b, u)
            return 0

        lax.fori_loop(0, nblk // NS, rnd, 0)
        for s in range(NS):
            wait_out(s)

    return gather3


def _local_msgs(f_bond_p, w_localT):
    def body(fb_ref, w_ref, loc_ref, msg_ref):
        loc = jnp.dot(fb_ref[:], w_ref[:], preferred_element_type=jnp.float32)
        loc_ref[:] = loc
        msg_ref[:] = _elu(loc)

    return pl.pallas_call(
        body,
        grid=(NBP // BM,),
        in_specs=[pl.BlockSpec((BM, 8), lambda i: (i, 0)),
                  pl.BlockSpec((8, H), lambda i: (0, 0))],
        out_specs=[pl.BlockSpec((BM, H), lambda i: (i, 0))] * 2,
        out_shape=[jax.ShapeDtypeStruct((NBP, H), jnp.float32)] * 2,
    )(f_bond_p, w_localT)


def _msg_update_half(local, nei_half, w_msgT, off, dst=None):
    nbl = HALF // BM

    if dst is None:
        def body(l_ref, n_ref, w_ref, o_ref):
            o_ref[:] = _elu(l_ref[:] + jnp.dot(n_ref[:], w_ref[:],
                                               preferred_element_type=jnp.float32))

        return pl.pallas_call(
            body,
            grid=(nbl,),
            in_specs=[pl.BlockSpec((BM, H), lambda i: (i + off, 0)),
                      pl.BlockSpec((BM, H), lambda i: (i, 0)),
                      pl.BlockSpec((H, H), lambda i: (0, 0))],
            out_specs=pl.BlockSpec((BM, H), lambda i: (i + off, 0)),
            out_shape=jax.ShapeDtypeStruct((NBP, H), jnp.float32),
        )(local, nei_half, w_msgT)

    def body2(d_ref, l_ref, n_ref, w_ref, o_ref):
        o_ref[:] = _elu(l_ref[:] + jnp.dot(n_ref[:], w_ref[:],
                                           preferred_element_type=jnp.float32))

    return pl.pallas_call(
        body2,
        grid=(nbl,),
        in_specs=[pl.BlockSpec(memory_space=pltpu.MemorySpace.HBM),
                  pl.BlockSpec((BM, H), lambda i: (i + off, 0)),
                  pl.BlockSpec((BM, H), lambda i: (i, 0)),
                  pl.BlockSpec((H, H), lambda i: (0, 0))],
        out_specs=pl.BlockSpec((BM, H), lambda i: (i + off, 0)),
        out_shape=jax.ShapeDtypeStruct((NBP, H), jnp.float32),
        input_output_aliases={0: 0},
    )(dst, local, nei_half, w_msgT)


def _node_emb(f64, wA, nb, wB):
    def body(f_ref, wa_ref, n_ref, wb_ref, o_ref):
        o_ref[:] = _elu(jnp.dot(f_ref[:], wa_ref[:], preferred_element_type=jnp.float32)
                        + jnp.dot(n_ref[:], wb_ref[:], preferred_element_type=jnp.float32))

    return pl.pallas_call(
        body,
        out_shape=jax.ShapeDtypeStruct((64, H), jnp.float32),
    )(f64, wA, nb, wB)


def _lstm(emb, Pf, Pb, wihtf, whhtf, bf, wihtb, whhtb, bb, Sf, Sb):
    def body(emb_ref, pf_ref, pb_ref, wif_ref, whf_ref, bf_ref,
             wib_ref, whb_ref, bb_ref, sf_ref, sb_ref,
             nuc_ref, gv_ref, xwf_s, xwb_s, of_s, ob_s):
        xf = jnp.dot(pf_ref[:], emb_ref[:], preferred_element_type=jnp.float32)
        xwf_s[:] = jnp.dot(xf, wif_ref[:], preferred_element_type=jnp.float32) + bf_ref[0:1, :]
        xb = jnp.dot(pb_ref[:], emb_ref[:], preferred_element_type=jnp.float32)
        xwb_s[:] = jnp.dot(xb, wib_ref[:], preferred_element_type=jnp.float32) + bb_ref[0:1, :]
        b_iota = lax.broadcasted_iota(jnp.int32, (BATCH, 1), 0)

        def step(t, carry):
            hf, cf, hb, cb = carry
            valid = t <= 2 * b_iota
            gf = xwf_s[pl.ds(t * BATCH, BATCH), :] + jnp.dot(
                hf, whf_ref[:], preferred_element_type=jnp.float32)
            i = jax.nn.sigmoid(gf[:, :Hh])
            f = jax.nn.sigmoid(gf[:, Hh:2 * Hh])
            g = jnp.tanh(gf[:, 2 * Hh:3 * Hh])
            o = jax.nn.sigmoid(gf[:, 3 * Hh:])
            cfn = f * cf + i * g
            hfn = o * jnp.tanh(cfn)
            of_s[pl.ds(t * BATCH, BATCH), :] = jnp.where(valid, hfn, 0.0)
            hf = jnp.where(valid, hfn, hf)
            cf = jnp.where(valid, cfn, cf)
            gb = xwb_s[pl.ds(t * BATCH, BATCH), :] + jnp.dot(
                hb, whb_ref[:], preferred_element_type=jnp.float32)
            ib = jax.nn.sigmoid(gb[:, :Hh])
            fb = jax.nn.sigmoid(gb[:, Hh:2 * Hh])
            gg = jnp.tanh(gb[:, 2 * Hh:3 * Hh])
            ob = jax.nn.sigmoid(gb[:, 3 * Hh:])
            cbn = fb * cb + ib * gg
            hbn = ob * jnp.tanh(cbn)
            ob_s[pl.ds(t * BATCH, BATCH), :] = jnp.where(valid, hbn, 0.0)
            hb = jnp.where(valid, hbn, hb)
            cb = jnp.where(valid, cbn, cb)
            return (hf, cf, hb, cb)

        z = jnp.zeros((BATCH, Hh), jnp.float32)
        hf, cf, hb, cb = lax.fori_loop(0, T, step, (z, z, z, z))
        nuc_ref[:, :Hh] = jnp.dot(sf_ref[:], of_s[:], preferred_element_type=jnp.float32)
        nuc_ref[:, Hh:] = jnp.dot(sb_ref[:], ob_s[:], preferred_element_type=jnp.float32)
        gv_ref[:, :Hh] = hf
        gv_ref[:, Hh:] = hb

    return pl.pallas_call(
        body,
        out_shape=[jax.ShapeDtypeStruct((256, H), jnp.float32),
                   jax.ShapeDtypeStruct((BATCH, H), jnp.float32)],
        scratch_shapes=[pltpu.VMEM((TB, 4 * Hh), jnp.float32),
                        pltpu.VMEM((TB, 4 * Hh), jnp.float32),
                        pltpu.VMEM((TB, Hh), jnp.float32),
                        pltpu.VMEM((TB, Hh), jnp.float32)],
    )(emb, Pf, Pb, wihtf, whhtf, bf, wihtb, whhtb, bb, Sf, Sb)


def _onehots():
    Pf = np.zeros((TB, 64), np.float32)
    Pb = np.zeros((TB, 64), np.float32)
    Sf = np.zeros((256, TB), np.float32)
    Sb = np.zeros((256, TB), np.float32)
    for b in range(BATCH):
        L = 2 * b + 1
        for t in range(L):
            Pf[t * BATCH + b, 2 * b + t] = 1
            Pb[t * BATCH + b, 4 * b - t] = 1
            Sf[b * b + t, t * BATCH + b] = 1
            Sb[b * b + t, (2 * b - t) * BATCH + b] = 1
    return Pf, Pb, Sf, Sb


_PF, _PB, _SF, _SB = _onehots()


def kernel(f_nuc, f_bond, node_graph, message_graph, all_bonds, scope,
           w_local, w_msg, w_node_emb, w_ih_f, w_hh_f, b_ih_f, b_hh_f,
           w_ih_b, w_hh_b, b_ih_b, b_hh_b):
    f_bond_p = jnp.pad(f_bond, ((0, NBP - NB), (0, 0)))
    mg = jnp.pad(message_graph.astype(jnp.int32), ((0, NBP - NB), (0, 0)))
    idx_a = mg[:HALF].T.reshape(3, 32, HALF // 32).transpose(1, 0, 2)
    idx_b = mg[HALF:].T.reshape(3, 32, HALF // 32).transpose(1, 0, 2)

    local, msgs = _local_msgs(f_bond_p, w_local.T)
    w_msgT = w_msg.T
    gat = _make_gather3(HALF, HALF // 32, G)
    zero_tok = jnp.zeros((8, 8), jnp.float32)
    for _ in range(DEPTH - 1):
        nei_a = gat(msgs, idx_a, zero_tok)
        nei_b = gat(msgs, idx_b, lax.slice(nei_a, (0, 0), (8, 8)))
        ma = _msg_update_half(local, nei_a, w_msgT, 0)
        msgs = _msg_update_half(local, nei_b, w_msgT, HALF // BM, dst=ma)

    ng = jnp.pad(node_graph[:64].astype(jnp.int32), ((0, 256 - 64), (0, 0)))
    idx_n = ng.T.reshape(3, 32, 8).transpose(1, 0, 2)
    nbr = _make_gather3(256, 8, 8)(msgs, idx_n)[:64]

    f64 = jnp.pad(f_nuc[:64], ((0, 0), (0, 4)))
    wA = jnp.pad(w_node_emb[:, :4].T, ((0, 4), (0, 0)))
    wB = w_node_emb[:, 4:].T
    emb = _node_emb(f64, wA, nbr, wB)

    bf = jnp.broadcast_to((b_ih_f + b_hh_f)[None, :], (8, 4 * Hh))
    bb = jnp.broadcast_to((b_ih_b + b_hh_b)[None, :], (8, 4 * Hh))
    nuc_out, gvec = _lstm(emb, jnp.asarray(_PF), jnp.asarray(_PB),
                          w_ih_f.T, w_hh_f.T, bf,
                          w_ih_b.T, w_hh_b.T, bb,
                          jnp.asarray(_SF), jnp.asarray(_SB))
    return nuc_out, gvec


# speedup vs baseline: 1.1875x; 1.1875x over previous
import functools

import numpy as np
import jax
import jax.numpy as jnp
from jax import lax
from jax.experimental import pallas as pl
from jax.experimental.pallas import tpu as pltpu
from jax.experimental.pallas import tpu_sc as plsc

H = 1024
Hh = 512
NB = 60000
NBP = 61440
RPW = 1920
G = 16
DEPTH = 12
BM = 512
HALF = NBP // 2
T = 31
BATCH = 16
TB = T * BATCH


def _elu(x):
    return jnp.where(x > 0, x, jnp.exp(jnp.minimum(x, 0.0)) - 1.0)


def _make_gather3(n_pad, rpw, g):
    nblk = rpw // g
    assert g % 8 == 0
    mesh = plsc.VectorSubcoreMesh(core_axis_name="c", subcore_axis_name="s")

    def _adds(r0, r1, r2):
        def row(j, _):
            for cb in range(H // 16):
                s = cb * 16
                r0[j, pl.ds(s, 16)] = (r0[j, pl.ds(s, 16)]
                                       + r1[j, pl.ds(s, 16)]
                                       + r2[j, pl.ds(s, 16)])
            return 0
        lax.fori_loop(0, g, row, 0)

    if nblk == 1:
        @functools.partial(
            pl.kernel, mesh=mesh,
            out_type=jax.ShapeDtypeStruct((n_pad, H), jnp.float32),
            scratch_types=[pltpu.VMEM((3, rpw), jnp.int32)]
            + [pltpu.VMEM((g, H), jnp.float32)] * 3
            + [pltpu.SemaphoreType.DMA],
        )
        def gather3_small(table_hbm, idx_hbm, out_hbm, idx_v, r0, r1, r2, sem):
            wid = lax.axis_index("s") * 2 + lax.axis_index("c")
            pltpu.sync_copy(idx_hbm.at[wid], idx_v)
            for k, r in ((0, r0), (1, r1), (2, r2)):
                pltpu.async_copy(table_hbm.at[idx_v.at[k, pl.ds(0, g)]], r, sem)
            for k, r in ((0, r0), (1, r1), (2, r2)):
                pltpu.make_async_copy(table_hbm.at[idx_v.at[k, pl.ds(0, g)]], r, sem).wait()
            _adds(r0, r1, r2)
            pltpu.sync_copy(r0, out_hbm.at[pl.ds(wid * rpw, g)])

        return gather3_small

    assert nblk % 2 == 0

    @functools.partial(
        pl.kernel, mesh=mesh,
        out_type=jax.ShapeDtypeStruct((n_pad, H), jnp.float32),
        scratch_types=[pltpu.VMEM((3, rpw), jnp.int32)]
        + [pltpu.VMEM((g, H), jnp.float32)] * 6
        + [pltpu.SemaphoreType.DMA] * 4,
    )
    def gather3(table_hbm, idx_hbm, tok_hbm, out_hbm, idx_v,
                a0, a1, a2, b0, b1, b2, sg0, sg1, so0, so1):
        del tok_hbm
        wid = lax.axis_index("s") * 2 + lax.axis_index("c")
        pltpu.sync_copy(idx_hbm.at[wid], idx_v)
        bufs = ((a0, a1, a2, sg0, so0), (b0, b1, b2, sg1, so1))

        def fire(bi, s):
            r0, r1, r2, sem, _ = bufs[s]
            for k, r in ((0, r0), (1, r1), (2, r2)):
                pltpu.async_copy(table_hbm.at[idx_v.at[k, pl.ds(bi * g, g)]], r, sem)

        def drain(s):
            r0, r1, r2, sem, _ = bufs[s]
            for k, r in ((0, r0), (1, r1), (2, r2)):
                pltpu.make_async_copy(table_hbm.at[idx_v.at[k, pl.ds(0, g)]], r, sem).wait()

        def wait_out(s):
            r0, _, _, _, osem = bufs[s]
            pltpu.make_async_copy(r0, out_hbm.at[pl.ds(wid * rpw, g)], osem).wait()

        def emit(bi, s):
            drain(s)

            @pl.when(bi >= 2)
            def _():
                wait_out(s)

            r0, r1, r2, _, osem = bufs[s]
            _adds(r0, r1, r2)
            pltpu.async_copy(r0, out_hbm.at[pl.ds(wid * rpw + bi * g, g)], osem)

        fire(0, 0)

        def pair(b2, _):
            b = b2 * 2
            fire(b + 1, 1)
            emit(b, 0)

            @pl.when(b2 < nblk // 2 - 1)
            def _():
                fire(b + 2, 0)

            emit(b + 1, 1)
            return 0

        lax.fori_loop(0, nblk // 2, pair, 0)
        wait_out(0)
        wait_out(1)

    return gather3


def _local_msgs(f_bond_p, w_localT):
    def body(fb_ref, w_ref, loc_ref, msg_ref):
        loc = jnp.dot(fb_ref[:], w_ref[:], preferred_element_type=jnp.float32)
        loc_ref[:] = loc
        msg_ref[:] = _elu(loc)

    return pl.pallas_call(
        body,
        grid=(NBP // BM,),
        in_specs=[pl.BlockSpec((BM, 8), lambda i: (i, 0)),
                  pl.BlockSpec((8, H), lambda i: (0, 0))],
        out_specs=[pl.BlockSpec((BM, H), lambda i: (i, 0))] * 2,
        out_shape=[jax.ShapeDtypeStruct((NBP, H), jnp.float32)] * 2,
    )(f_bond_p, w_localT)


def _msg_update_half(local, nei_half, w_msgT, off, dst=None):
    nbl = nei_half.shape[0] // BM

    if dst is None:
        def body(l_ref, n_ref, w_ref, o_ref):
            o_ref[:] = _elu(l_ref[:] + jnp.dot(n_ref[:], w_ref[:],
                                               preferred_element_type=jnp.float32))

        return pl.pallas_call(
            body,
            grid=(nbl,),
            in_specs=[pl.BlockSpec((BM, H), lambda i: (i + off, 0)),
                      pl.BlockSpec((BM, H), lambda i: (i, 0)),
                      pl.BlockSpec((H, H), lambda i: (0, 0))],
            out_specs=pl.BlockSpec((BM, H), lambda i: (i + off, 0)),
            out_shape=jax.ShapeDtypeStruct((NBP, H), jnp.float32),
        )(local, nei_half, w_msgT)

    def body2(d_ref, l_ref, n_ref, w_ref, o_ref):
        o_ref[:] = _elu(l_ref[:] + jnp.dot(n_ref[:], w_ref[:],
                                           preferred_element_type=jnp.float32))

    return pl.pallas_call(
        body2,
        grid=(nbl,),
        in_specs=[pl.BlockSpec(memory_space=pltpu.MemorySpace.HBM),
                  pl.BlockSpec((BM, H), lambda i: (i + off, 0)),
                  pl.BlockSpec((BM, H), lambda i: (i, 0)),
                  pl.BlockSpec((H, H), lambda i: (0, 0))],
        out_specs=pl.BlockSpec((BM, H), lambda i: (i + off, 0)),
        out_shape=jax.ShapeDtypeStruct((NBP, H), jnp.float32),
        input_output_aliases={0: 0},
    )(dst, local, nei_half, w_msgT)


def _node_emb(f64, wA, nb, wB):
    def body(f_ref, wa_ref, n_ref, wb_ref, o_ref):
        o_ref[:] = _elu(jnp.dot(f_ref[:], wa_ref[:], preferred_element_type=jnp.float32)
                        + jnp.dot(n_ref[:], wb_ref[:], preferred_element_type=jnp.float32))

    return pl.pallas_call(
        body,
        out_shape=jax.ShapeDtypeStruct((64, H), jnp.float32),
    )(f64, wA, nb, wB)


def _lstm(emb, Pf, Pb, wihtf, whhtf, bf, wihtb, whhtb, bb, Sf, Sb):
    def body(emb_ref, pf_ref, pb_ref, wif_ref, whf_ref, bf_ref,
             wib_ref, whb_ref, bb_ref, sf_ref, sb_ref,
             nuc_ref, gv_ref, xwf_s, xwb_s, of_s, ob_s):
        xf = jnp.dot(pf_ref[:], emb_ref[:], preferred_element_type=jnp.float32)
        xwf_s[:] = jnp.dot(xf, wif_ref[:], preferred_element_type=jnp.float32) + bf_ref[0:1, :]
        xb = jnp.dot(pb_ref[:], emb_ref[:], preferred_element_type=jnp.float32)
        xwb_s[:] = jnp.dot(xb, wib_ref[:], preferred_element_type=jnp.float32) + bb_ref[0:1, :]
        b_iota = lax.broadcasted_iota(jnp.int32, (BATCH, 1), 0)

        def step(t, carry):
            hf, cf, hb, cb = carry
            valid = t <= 2 * b_iota
            gf = xwf_s[pl.ds(t * BATCH, BATCH), :] + jnp.dot(
                hf, whf_ref[:], preferred_element_type=jnp.float32)
            i = jax.nn.sigmoid(gf[:, :Hh])
            f = jax.nn.sigmoid(gf[:, Hh:2 * Hh])
            g = jnp.tanh(gf[:, 2 * Hh:3 * Hh])
            o = jax.nn.sigmoid(gf[:, 3 * Hh:])
            cfn = f * cf + i * g
            hfn = o * jnp.tanh(cfn)
            of_s[pl.ds(t * BATCH, BATCH), :] = jnp.where(valid, hfn, 0.0)
            hf = jnp.where(valid, hfn, hf)
            cf = jnp.where(valid, cfn, cf)
            gb = xwb_s[pl.ds(t * BATCH, BATCH), :] + jnp.dot(
                hb, whb_ref[:], preferred_element_type=jnp.float32)
            ib = jax.nn.sigmoid(gb[:, :Hh])
            fb = jax.nn.sigmoid(gb[:, Hh:2 * Hh])
            gg = jnp.tanh(gb[:, 2 * Hh:3 * Hh])
            ob = jax.nn.sigmoid(gb[:, 3 * Hh:])
            cbn = fb * cb + ib * gg
            hbn = ob * jnp.tanh(cbn)
            ob_s[pl.ds(t * BATCH, BATCH), :] = jnp.where(valid, hbn, 0.0)
            hb = jnp.where(valid, hbn, hb)
            cb = jnp.where(valid, cbn, cb)
            return (hf, cf, hb, cb)

        z = jnp.zeros((BATCH, Hh), jnp.float32)
        hf, cf, hb, cb = lax.fori_loop(0, T, step, (z, z, z, z))
        nuc_ref[:, :Hh] = jnp.dot(sf_ref[:], of_s[:], preferred_element_type=jnp.float32)
        nuc_ref[:, Hh:] = jnp.dot(sb_ref[:], ob_s[:], preferred_element_type=jnp.float32)
        gv_ref[:, :Hh] = hf
        gv_ref[:, Hh:] = hb

    return pl.pallas_call(
        body,
        out_shape=[jax.ShapeDtypeStruct((256, H), jnp.float32),
                   jax.ShapeDtypeStruct((BATCH, H), jnp.float32)],
        scratch_shapes=[pltpu.VMEM((TB, 4 * Hh), jnp.float32),
                        pltpu.VMEM((TB, 4 * Hh), jnp.float32),
                        pltpu.VMEM((TB, Hh), jnp.float32),
                        pltpu.VMEM((TB, Hh), jnp.float32)],
    )(emb, Pf, Pb, wihtf, whhtf, bf, wihtb, whhtb, bb, Sf, Sb)


def _onehots():
    Pf = np.zeros((TB, 64), np.float32)
    Pb = np.zeros((TB, 64), np.float32)
    Sf = np.zeros((256, TB), np.float32)
    Sb = np.zeros((256, TB), np.float32)
    for b in range(BATCH):
        L = 2 * b + 1
        for t in range(L):
            Pf[t * BATCH + b, 2 * b + t] = 1
            Pb[t * BATCH + b, 4 * b - t] = 1
            Sf[b * b + t, t * BATCH + b] = 1
            Sb[b * b + t, (2 * b - t) * BATCH + b] = 1
    return Pf, Pb, Sf, Sb


_PF, _PB, _SF, _SB = _onehots()


def kernel(f_nuc, f_bond, node_graph, message_graph, all_bonds, scope,
           w_local, w_msg, w_node_emb, w_ih_f, w_hh_f, b_ih_f, b_hh_f,
           w_ih_b, w_hh_b, b_ih_b, b_hh_b):
    f_bond_p = jnp.pad(f_bond, ((0, NBP - NB), (0, 0)))
    mg = jnp.pad(message_graph.astype(jnp.int32), ((0, NBP - NB), (0, 0)))
    THIRD = NBP // 3
    idx_s = [mg[i * THIRD:(i + 1) * THIRD].T
             .reshape(3, 32, THIRD // 32).transpose(1, 0, 2) for i in range(3)]

    local, msgs = _local_msgs(f_bond_p, w_local.T)
    w_msgT = w_msg.T
    gat = _make_gather3(THIRD, THIRD // 32, G)
    zero_tok = jnp.zeros((8, 8), jnp.float32)
    for _ in range(DEPTH - 1):
        neis = [gat(msgs, ix, zero_tok) for ix in idx_s]
        dst = None
        for i, nei in enumerate(neis):
            dst = _msg_update_half(local, nei, w_msgT, i * (THIRD // BM), dst=dst)
        msgs = dst

    ng = jnp.pad(node_graph[:64].astype(jnp.int32), ((0, 256 - 64), (0, 0)))
    idx_n = ng.T.reshape(3, 32, 8).transpose(1, 0, 2)
    nbr = _make_gather3(256, 8, 8)(msgs, idx_n)[:64]

    f64 = jnp.pad(f_nuc[:64], ((0, 0), (0, 4)))
    wA = jnp.pad(w_node_emb[:, :4].T, ((0, 4), (0, 0)))
    wB = w_node_emb[:, 4:].T
    emb = _node_emb(f64, wA, nbr, wB)

    bf = jnp.broadcast_to((b_ih_f + b_hh_f)[None, :], (8, 4 * Hh))
    bb = jnp.broadcast_to((b_ih_b + b_hh_b)[None, :], (8, 4 * Hh))
    nuc_out, gvec = _lstm(emb, jnp.asarray(_PF), jnp.asarray(_PB),
                          w_ih_f.T, w_hh_f.T, bf,
                          w_ih_b.T, w_hh_b.T, bb,
                          jnp.asarray(_SF), jnp.asarray(_SB))
    return nuc_out, gvec
